# Initial kernel scaffold; baseline (speedup 1.0000x reference)
import jax
import jax.numpy as jnp
from jax.experimental import pallas as pl


def kernel(anchors, objectness, pred_bbox_deltas):
    raise NotImplementedError("write your pallas kernel here")


# speedup vs baseline: 11.1823x; 11.1823x over previous
import jax
import jax.numpy as jnp
from jax import lax
from jax.experimental import pallas as pl
from jax.experimental.pallas import tpu as pltpu
import numpy as np

_N = 20000
_NP = 20480
_PRE = 2000
_PAD = 2048
_POST = 1000
_THR = 0.7
_MIN_SIZE = 1e-3
_IMG_H, _IMG_W = 800.0, 1333.0
_CLIP = float(np.log(1000.0 / 16.0))
_NEG = -1e9
_NB = 16


def _decode_body(a_ref, d_ref, o_ref, b_ref, s_ref):
    ax1 = a_ref[0:1, :]
    ay1 = a_ref[1:2, :]
    ax2 = a_ref[2:3, :]
    ay2 = a_ref[3:4, :]
    w = ax2 - ax1
    h = ay2 - ay1
    cx = ax1 + 0.5 * w
    cy = ay1 + 0.5 * h
    dx = d_ref[0:1, :]
    dy = d_ref[1:2, :]
    dw = jnp.minimum(d_ref[2:3, :], _CLIP)
    dh = jnp.minimum(d_ref[3:4, :], _CLIP)
    pcx = dx * w + cx
    pcy = dy * h + cy
    pw = jnp.exp(dw) * w
    ph = jnp.exp(dh) * h
    x1 = jnp.clip(pcx - 0.5 * pw, 0.0, _IMG_W)
    y1 = jnp.clip(pcy - 0.5 * ph, 0.0, _IMG_H)
    x2 = jnp.clip(pcx + 0.5 * pw, 0.0, _IMG_W)
    y2 = jnp.clip(pcy + 0.5 * ph, 0.0, _IMG_H)
    valid = ((x2 - x1) >= _MIN_SIZE) & ((y2 - y1) >= _MIN_SIZE)
    b_ref[0:1, :] = x1
    b_ref[1:2, :] = y1
    b_ref[2:3, :] = x2
    b_ref[3:4, :] = y2
    b_ref[4:8, :] = jnp.zeros((4, _NP), jnp.float32)
    s_ref[0:1, :] = jnp.where(valid, o_ref[0:1, :], _NEG)


def _coords_col(col_ref, base):
    blk = col_ref[pl.ds(base, 128), :]
    return blk[:, 0:1], blk[:, 1:2], blk[:, 2:3], blk[:, 3:4]


def _coords_row(blk_ref, i):
    def one(c):
        return blk_ref[c : c + 1, pl.ds(i, 1), :].reshape(1, 128)

    return one(0), one(1), one(2), one(3)


def _iou_tile(rc, cc):
    rx1, ry1, rx2, ry2 = rc
    cx1, cy1, cx2, cy2 = cc
    ra = (rx2 - rx1) * (ry2 - ry1)
    ca = (cx2 - cx1) * (cy2 - cy1)
    iw = jnp.maximum(jnp.minimum(rx2, cx2) - jnp.maximum(rx1, cx1), 0.0)
    ih = jnp.maximum(jnp.minimum(ry2, cy2) - jnp.maximum(ry1, cy1), 0.0)
    inter = iw * ih
    return inter / (ra + ca - inter)


def _nms_body(col_ref, blk_ref, sc_ref, keep_ref, msc_ref, iou_s):
    lane = lax.broadcasted_iota(jnp.int32, (1, 128), 1)

    def outer(i, _):
        ci = _coords_row(blk_ref, i)

        def cross(b, acc):
            rb = _coords_col(col_ref, b * 128)
            s_bi = jnp.where(_iou_tile(rb, ci) > _THR, 1.0, 0.0)
            kb = jnp.where(b < i, keep_ref[pl.ds(b, 1), :], 0.0)
            return acc + jnp.dot(kb, s_bi, preferred_element_type=jnp.float32)

        acc = lax.fori_loop(0, _NB, cross, jnp.zeros((1, 128), jnp.float32))
        keep_blk = jnp.where(acc > 0.0, 0.0, 1.0)

        ri = _coords_col(col_ref, i * 128)
        iou_s[:, :] = _iou_tile(ri, ci)

        def inner(t, kb):
            row_t = iou_s[pl.ds(t, 1), :]
            k_t = jnp.max(jnp.where(lane == t, kb, 0.0))
            sup = jnp.where((row_t > _THR) & (lane > t), 1.0, 0.0)
            return kb * (1.0 - sup * k_t)

        keep_blk = lax.fori_loop(0, 128, inner, keep_blk)
        keep_ref[pl.ds(i, 1), :] = keep_blk
        msc_ref[pl.ds(i, 1), :] = jnp.where(
            keep_blk > 0.5, sc_ref[pl.ds(i, 1), :], _NEG
        )
        return 0

    lax.fori_loop(0, _NB, outer, 0)


def kernel(anchors, objectness, pred_bbox_deltas):
    f32 = jnp.float32
    anchors8 = jnp.zeros((8, _NP), f32).at[0:4, :_N].set(anchors.T)
    deltas8 = jnp.zeros((8, _NP), f32).at[0:4, :_N].set(pred_bbox_deltas.T)
    obj = jnp.zeros((1, _NP), f32).at[0, :_N].set(objectness)

    boxes8, scores = pl.pallas_call(
        _decode_body,
        out_shape=(
            jax.ShapeDtypeStruct((8, _NP), f32),
            jax.ShapeDtypeStruct((1, _NP), f32),
        ),
    )(anchors8, deltas8, obj)

    top_scores, top_idx = lax.top_k(scores[0, :_N], _PRE)
    tb = boxes8[0:4, :][:, top_idx]
    boxes_p = jnp.pad(tb, ((0, 4), (0, _PAD - _PRE)))
    boxes_col = boxes_p.T[:, 0:4]
    boxes_blk = boxes_p[0:4].reshape(4, _NB, 128)
    scores_blk = (
        jnp.full((_PAD,), _NEG, f32).at[:_PRE].set(top_scores).reshape(_NB, 128)
    )

    _, msc = pl.pallas_call(
        _nms_body,
        out_shape=(
            jax.ShapeDtypeStruct((_NB, 128), f32),
            jax.ShapeDtypeStruct((_NB, 128), f32),
        ),
        scratch_shapes=[pltpu.VMEM((128, 128), f32)],
    )(boxes_col, boxes_blk, scores_blk)

    sel_scores, sel_idx = lax.top_k(msc.reshape(_PAD)[:_PRE], _POST)
    final_boxes = tb.T[sel_idx]
    return final_boxes, sel_scores


# speedup vs baseline: 27.2363x; 2.4357x over previous
import jax
import jax.numpy as jnp
from jax import lax
from jax.experimental import pallas as pl
from jax.experimental.pallas import tpu as pltpu
import numpy as np

_N = 20000
_NP = 20480
_PRE = 2000
_PAD = 2048
_POST = 1000
_THR = 0.7
_MIN_SIZE = 1e-3
_IMG_H, _IMG_W = 800.0, 1333.0
_CLIP = float(np.log(1000.0 / 16.0))
_NEG = -1e9
_NB = 16


def _decode_body(a_ref, d_ref, o_ref, b_ref, s_ref):
    ax1 = a_ref[0:1, :]
    ay1 = a_ref[1:2, :]
    ax2 = a_ref[2:3, :]
    ay2 = a_ref[3:4, :]
    w = ax2 - ax1
    h = ay2 - ay1
    cx = ax1 + 0.5 * w
    cy = ay1 + 0.5 * h
    dx = d_ref[0:1, :]
    dy = d_ref[1:2, :]
    dw = jnp.minimum(d_ref[2:3, :], _CLIP)
    dh = jnp.minimum(d_ref[3:4, :], _CLIP)
    pcx = dx * w + cx
    pcy = dy * h + cy
    pw = jnp.exp(dw) * w
    ph = jnp.exp(dh) * h
    x1 = jnp.clip(pcx - 0.5 * pw, 0.0, _IMG_W)
    y1 = jnp.clip(pcy - 0.5 * ph, 0.0, _IMG_H)
    x2 = jnp.clip(pcx + 0.5 * pw, 0.0, _IMG_W)
    y2 = jnp.clip(pcy + 0.5 * ph, 0.0, _IMG_H)
    valid = ((x2 - x1) >= _MIN_SIZE) & ((y2 - y1) >= _MIN_SIZE)
    b_ref[0:1, :] = x1
    b_ref[1:2, :] = y1
    b_ref[2:3, :] = x2
    b_ref[3:4, :] = y2
    b_ref[4:8, :] = jnp.zeros((4, _NP), jnp.float32)
    s_ref[0:1, :] = jnp.where(valid, o_ref[0:1, :], _NEG)


def _coords_col(col_ref, base):
    blk = col_ref[pl.ds(base, 128), :]
    return blk[:, 0:1], blk[:, 1:2], blk[:, 2:3], blk[:, 3:4]


def _coords_row(blk_ref, i):
    def one(c):
        return blk_ref[c : c + 1, pl.ds(i, 1), :].reshape(1, 128)

    return one(0), one(1), one(2), one(3)


def _iou_tile(rc, cc):
    rx1, ry1, rx2, ry2 = rc
    cx1, cy1, cx2, cy2 = cc
    ra = (rx2 - rx1) * (ry2 - ry1)
    ca = (cx2 - cx1) * (cy2 - cy1)
    iw = jnp.maximum(jnp.minimum(rx2, cx2) - jnp.maximum(rx1, cx1), 0.0)
    ih = jnp.maximum(jnp.minimum(ry2, cy2) - jnp.maximum(ry1, cy1), 0.0)
    inter = iw * ih
    return inter / (ra + ca - inter)


def _nms_body(col_ref, blk_ref, sc_ref, keep_ref, msc_ref):
    lane = lax.broadcasted_iota(jnp.int32, (1, 128), 1)
    sub = lax.broadcasted_iota(jnp.int32, (128, 1), 0)

    def outer(i, _):
        ci = _coords_row(blk_ref, i)

        def cross(b, acc):
            rb = _coords_col(col_ref, b * 128)
            s_bi = jnp.where(_iou_tile(rb, ci) > _THR, 1.0, 0.0)
            kb = jnp.where(b < i, keep_ref[pl.ds(b, 1), :], 0.0)
            return acc + jnp.dot(kb, s_bi, preferred_element_type=jnp.float32)

        acc = lax.fori_loop(0, _NB, cross, jnp.zeros((1, 128), jnp.float32))
        kbc = jnp.where(acc > 0.0, 0.0, 1.0)

        ri = _coords_col(col_ref, i * 128)
        s_low = jnp.where((_iou_tile(ri, ci) > _THR) & (sub < lane), 1.0, 0.0)

        def fix_cond(c):
            return c[1]

        def fix_body(c):
            kb, _ = c
            sup = jnp.dot(kb, s_low, preferred_element_type=jnp.float32)
            nkb = jnp.where(sup > 0.0, 0.0, kbc)
            return nkb, jnp.any(nkb != kb)

        keep_blk, _ = lax.while_loop(fix_cond, fix_body, (kbc, True))
        keep_ref[pl.ds(i, 1), :] = keep_blk
        msc_ref[pl.ds(i, 1), :] = jnp.where(
            keep_blk > 0.5, sc_ref[pl.ds(i, 1), :], _NEG
        )
        return 0

    lax.fori_loop(0, _NB, outer, 0)


def kernel(anchors, objectness, pred_bbox_deltas):
    f32 = jnp.float32
    anchors8 = jnp.zeros((8, _NP), f32).at[0:4, :_N].set(anchors.T)
    deltas8 = jnp.zeros((8, _NP), f32).at[0:4, :_N].set(pred_bbox_deltas.T)
    obj = jnp.zeros((1, _NP), f32).at[0, :_N].set(objectness)

    boxes8, scores = pl.pallas_call(
        _decode_body,
        out_shape=(
            jax.ShapeDtypeStruct((8, _NP), f32),
            jax.ShapeDtypeStruct((1, _NP), f32),
        ),
    )(anchors8, deltas8, obj)

    top_scores, top_idx = lax.top_k(scores[0, :_N], _PRE)
    tb = boxes8[0:4, :][:, top_idx]
    boxes_p = jnp.pad(tb, ((0, 4), (0, _PAD - _PRE)))
    boxes_col = boxes_p.T[:, 0:4]
    boxes_blk = boxes_p[0:4].reshape(4, _NB, 128)
    scores_blk = (
        jnp.full((_PAD,), _NEG, f32).at[:_PRE].set(top_scores).reshape(_NB, 128)
    )

    _, msc = pl.pallas_call(
        _nms_body,
        out_shape=(
            jax.ShapeDtypeStruct((_NB, 128), f32),
            jax.ShapeDtypeStruct((_NB, 128), f32),
        ),
    )(boxes_col, boxes_blk, scores_blk)

    sel_scores, sel_idx = lax.top_k(msc.reshape(_PAD)[:_PRE], _POST)
    final_boxes = tb.T[sel_idx]
    return final_boxes, sel_scores


# speedup vs baseline: 50.9219x; 1.8696x over previous
import jax
import jax.numpy as jnp
from jax import lax
from jax.experimental import pallas as pl
import numpy as np

_N = 20000
_NP = 20480
_PRE = 2000
_PAD = 2048
_POST = 1000
_THR = 0.7
_MIN_SIZE = 1e-3
_IMG_H, _IMG_W = 800.0, 1333.0
_CLIP = float(np.log(1000.0 / 16.0))
_NEG = -1e9
_BW = 256
_NB = _PAD // _BW


def _decode_body(a_ref, d_ref, o_ref, b_ref, s_ref):
    ax1 = a_ref[0:1, :]
    ay1 = a_ref[1:2, :]
    ax2 = a_ref[2:3, :]
    ay2 = a_ref[3:4, :]
    w = ax2 - ax1
    h = ay2 - ay1
    cx = ax1 + 0.5 * w
    cy = ay1 + 0.5 * h
    dx = d_ref[0:1, :]
    dy = d_ref[1:2, :]
    dw = jnp.minimum(d_ref[2:3, :], _CLIP)
    dh = jnp.minimum(d_ref[3:4, :], _CLIP)
    pcx = dx * w + cx
    pcy = dy * h + cy
    pw = jnp.exp(dw) * w
    ph = jnp.exp(dh) * h
    x1 = jnp.clip(pcx - 0.5 * pw, 0.0, _IMG_W)
    y1 = jnp.clip(pcy - 0.5 * ph, 0.0, _IMG_H)
    x2 = jnp.clip(pcx + 0.5 * pw, 0.0, _IMG_W)
    y2 = jnp.clip(pcy + 0.5 * ph, 0.0, _IMG_H)
    valid = ((x2 - x1) >= _MIN_SIZE) & ((y2 - y1) >= _MIN_SIZE)
    b_ref[0:1, :] = x1
    b_ref[1:2, :] = y1
    b_ref[2:3, :] = x2
    b_ref[3:4, :] = y2
    b_ref[4:8, :] = jnp.zeros((4, _NP), jnp.float32)
    s_ref[0:1, :] = jnp.where(valid, o_ref[0:1, :], _NEG)


def _coords_col(col_ref, base):
    blk = col_ref[pl.ds(base, _BW), :]
    return blk[:, 0:1], blk[:, 1:2], blk[:, 2:3], blk[:, 3:4]


def _coords_row(blk_ref, i):
    def one(c):
        return blk_ref[c : c + 1, pl.ds(i, 1), :].reshape(1, _BW)

    return one(0), one(1), one(2), one(3)


def _iou_tile(rc, cc):
    rx1, ry1, rx2, ry2 = rc
    cx1, cy1, cx2, cy2 = cc
    ra = (rx2 - rx1) * (ry2 - ry1)
    ca = (cx2 - cx1) * (cy2 - cy1)
    iw = jnp.maximum(jnp.minimum(rx2, cx2) - jnp.maximum(rx1, cx1), 0.0)
    ih = jnp.maximum(jnp.minimum(ry2, cy2) - jnp.maximum(ry1, cy1), 0.0)
    inter = iw * ih
    return inter / (ra + ca - inter)


def _nms_body(col_ref, blk_ref, sc_ref, keep_ref, msc_ref):
    lane = lax.broadcasted_iota(jnp.int32, (1, _BW), 1)
    sub = lax.broadcasted_iota(jnp.int32, (_BW, 1), 0)

    def outer(i, _):
        ci = _coords_row(blk_ref, i)

        def cross(b, acc):
            rb = _coords_col(col_ref, b * _BW)
            s_bi = jnp.where(_iou_tile(rb, ci) > _THR, 1.0, 0.0)
            kb = keep_ref[pl.ds(b, 1), :]
            return acc + jnp.dot(kb, s_bi, preferred_element_type=jnp.float32)

        acc = lax.fori_loop(0, i, cross, jnp.zeros((1, _BW), jnp.float32))
        kbc = jnp.where(acc > 0.0, 0.0, 1.0)

        ri = _coords_col(col_ref, i * _BW)
        s_low = jnp.where((_iou_tile(ri, ci) > _THR) & (sub < lane), 1.0, 0.0)

        def fix_cond(c):
            return c[1]

        def fix_body(c):
            kb, _ = c
            sup = jnp.dot(kb, s_low, preferred_element_type=jnp.float32)
            nkb = jnp.where(sup > 0.0, 0.0, kbc)
            return nkb, jnp.any(nkb != kb)

        keep_blk, _ = lax.while_loop(fix_cond, fix_body, (kbc, True))
        keep_ref[pl.ds(i, 1), :] = keep_blk
        msc_ref[pl.ds(i, 1), :] = jnp.where(
            keep_blk > 0.5, sc_ref[pl.ds(i, 1), :], _NEG
        )
        return 0

    lax.fori_loop(0, _NB, outer, 0)


def kernel(anchors, objectness, pred_bbox_deltas):
    f32 = jnp.float32
    anchors8 = jnp.zeros((8, _NP), f32).at[0:4, :_N].set(anchors.T)
    deltas8 = jnp.zeros((8, _NP), f32).at[0:4, :_N].set(pred_bbox_deltas.T)
    obj = jnp.zeros((1, _NP), f32).at[0, :_N].set(objectness)

    boxes8, scores = pl.pallas_call(
        _decode_body,
        out_shape=(
            jax.ShapeDtypeStruct((8, _NP), f32),
            jax.ShapeDtypeStruct((1, _NP), f32),
        ),
    )(anchors8, deltas8, obj)

    top_scores, top_idx = lax.top_k(scores[0, :_N], _PRE)
    tb = boxes8[0:4, :][:, top_idx]
    boxes_p = jnp.pad(tb, ((0, 4), (0, _PAD - _PRE)))
    boxes_col = boxes_p.T[:, 0:4]
    boxes_blk = boxes_p[0:4].reshape(4, _NB, _BW)
    scores_blk = (
        jnp.full((_PAD,), _NEG, f32).at[:_PRE].set(top_scores).reshape(_NB, _BW)
    )

    _, msc = pl.pallas_call(
        _nms_body,
        out_shape=(
            jax.ShapeDtypeStruct((_NB, _BW), f32),
            jax.ShapeDtypeStruct((_NB, _BW), f32),
        ),
    )(boxes_col, boxes_blk, scores_blk)

    sel_scores, sel_idx = lax.top_k(msc.reshape(_PAD)[:_PRE], _POST)
    final_boxes = tb.T[sel_idx]
    return final_boxes, sel_scores


# speedup vs baseline: 53.6676x; 1.0539x over previous
import jax
import jax.numpy as jnp
from jax import lax
from jax.experimental import pallas as pl
import numpy as np

_N = 20000
_NP = 20480
_PRE = 2000
_PAD = 2048
_POST = 1000
_THR = 0.7
_MIN_SIZE = 1e-3
_IMG_H, _IMG_W = 800.0, 1333.0
_CLIP = float(np.log(1000.0 / 16.0))
_NEG = -1e9
_BW = 256
_NB = _PAD // _BW


def _decode_body(a_ref, d_ref, o_ref, b_ref, s_ref):
    ax1 = a_ref[0:1, :]
    ay1 = a_ref[1:2, :]
    ax2 = a_ref[2:3, :]
    ay2 = a_ref[3:4, :]
    w = ax2 - ax1
    h = ay2 - ay1
    cx = ax1 + 0.5 * w
    cy = ay1 + 0.5 * h
    dx = d_ref[0:1, :]
    dy = d_ref[1:2, :]
    dw = jnp.minimum(d_ref[2:3, :], _CLIP)
    dh = jnp.minimum(d_ref[3:4, :], _CLIP)
    pcx = dx * w + cx
    pcy = dy * h + cy
    pw = jnp.exp(dw) * w
    ph = jnp.exp(dh) * h
    x1 = jnp.clip(pcx - 0.5 * pw, 0.0, _IMG_W)
    y1 = jnp.clip(pcy - 0.5 * ph, 0.0, _IMG_H)
    x2 = jnp.clip(pcx + 0.5 * pw, 0.0, _IMG_W)
    y2 = jnp.clip(pcy + 0.5 * ph, 0.0, _IMG_H)
    valid = ((x2 - x1) >= _MIN_SIZE) & ((y2 - y1) >= _MIN_SIZE)
    b_ref[0:1, :] = x1
    b_ref[1:2, :] = y1
    b_ref[2:3, :] = x2
    b_ref[3:4, :] = y2
    b_ref[4:5, :] = (x2 - x1) * (y2 - y1)
    b_ref[5:8, :] = jnp.zeros((3, _NP), jnp.float32)
    s_ref[0:1, :] = jnp.where(valid, o_ref[0:1, :], _NEG)


def _coords_col(col_ref, base):
    blk = col_ref[pl.ds(base, _BW), :]
    return blk[:, 0:1], blk[:, 1:2], blk[:, 2:3], blk[:, 3:4], blk[:, 4:5]


def _coords_row(blk_ref, i):
    def one(c):
        return blk_ref[c : c + 1, pl.ds(i, 1), :].reshape(1, _BW)

    return one(0), one(1), one(2), one(3), one(4)


def _iou_tile(rc, cc):
    rx1, ry1, rx2, ry2, ra = rc
    cx1, cy1, cx2, cy2, ca = cc
    iw = jnp.maximum(jnp.minimum(rx2, cx2) - jnp.maximum(rx1, cx1), 0.0)
    ih = jnp.maximum(jnp.minimum(ry2, cy2) - jnp.maximum(ry1, cy1), 0.0)
    inter = iw * ih
    return inter / (ra + ca - inter)


def _nms_body(col_ref, blk_ref, sc_ref, keep_ref, msc_ref):
    lane = lax.broadcasted_iota(jnp.int32, (1, _BW), 1)
    sub = lax.broadcasted_iota(jnp.int32, (_BW, 1), 0)

    def outer(i, _):
        ci = _coords_row(blk_ref, i)

        def cross(b, acc):
            rb = _coords_col(col_ref, b * _BW)
            s_bi = jnp.where(_iou_tile(rb, ci) > _THR, 1.0, 0.0)
            kb = keep_ref[pl.ds(b, 1), :]
            return acc + jnp.dot(kb, s_bi, preferred_element_type=jnp.float32)

        acc = lax.fori_loop(0, i, cross, jnp.zeros((1, _BW), jnp.float32))
        kbc = jnp.where(acc > 0.0, 0.0, 1.0)

        ri = _coords_col(col_ref, i * _BW)
        s_low = jnp.where((_iou_tile(ri, ci) > _THR) & (sub < lane), 1.0, 0.0)

        def fix_cond(c):
            return c[1]

        def fix_body(c):
            kb, _ = c
            sup = jnp.dot(kb, s_low, preferred_element_type=jnp.float32)
            nkb = jnp.where(sup > 0.0, 0.0, kbc)
            return nkb, jnp.any(nkb != kb)

        keep_blk, _ = lax.while_loop(fix_cond, fix_body, (kbc, True))
        keep_ref[pl.ds(i, 1), :] = keep_blk
        msc_ref[pl.ds(i, 1), :] = jnp.where(
            keep_blk > 0.5, sc_ref[pl.ds(i, 1), :], _NEG
        )
        return 0

    lax.fori_loop(0, _NB, outer, 0)


def kernel(anchors, objectness, pred_bbox_deltas):
    f32 = jnp.float32
    anchors8 = jnp.zeros((8, _NP), f32).at[0:4, :_N].set(anchors.T)
    deltas8 = jnp.zeros((8, _NP), f32).at[0:4, :_N].set(pred_bbox_deltas.T)
    obj = jnp.zeros((1, _NP), f32).at[0, :_N].set(objectness)

    boxes8, scores = pl.pallas_call(
        _decode_body,
        out_shape=(
            jax.ShapeDtypeStruct((8, _NP), f32),
            jax.ShapeDtypeStruct((1, _NP), f32),
        ),
    )(anchors8, deltas8, obj)

    top_scores, top_idx = lax.top_k(scores[0, :_N], _PRE)
    tba = boxes8[0:5, :][:, top_idx]
    tb = tba[0:4]
    boxes_p = jnp.pad(tba, ((0, 3), (0, _PAD - _PRE)))
    boxes_col = boxes_p.T[:, 0:5]
    boxes_blk = boxes_p[0:5].reshape(5, _NB, _BW)
    scores_blk = (
        jnp.full((_PAD,), _NEG, f32).at[:_PRE].set(top_scores).reshape(_NB, _BW)
    )

    _, msc = pl.pallas_call(
        _nms_body,
        out_shape=(
            jax.ShapeDtypeStruct((_NB, _BW), f32),
            jax.ShapeDtypeStruct((_NB, _BW), f32),
        ),
    )(boxes_col, boxes_blk, scores_blk)

    sel_scores, sel_idx = lax.top_k(msc.reshape(_PAD)[:_PRE], _POST)
    final_boxes = tb.T[sel_idx]
    return final_boxes, sel_scores
